# baseline (speedup 1.0000x reference)
import jax
import jax.numpy as jnp
from jax import lax
from jax.experimental import pallas as pl
from jax.experimental.pallas import tpu as pltpu
from jax.experimental.pallas import tpu_sc as plsc

N_NODES = 10000
N_EDGES = 320000
D = 128
N_HOPS = 3

NC = 2
NS = 16
L = 16
DH = D // NC

CHUNK = 128
NCH = 160
BCH = 8
NBLK = NCH // BCH
EPT = NCH * CHUNK
EP = NS * EPT
NP = 10112
RPT = NP // NS


def _hop_body(agg_hbm, row_hbm, col_hbm, tr_hbm, out_hbm,
              table, acc, row_v, col_v, tr_v, gat_v, sem_i, sem_g, sem_s):
    c = lax.axis_index("c")
    s = lax.axis_index("s")
    cbase = s * NCH

    def idx_start(blk, b):
        r0 = cbase + blk * BCH
        pltpu.async_copy(row_hbm.at[pl.ds(r0, BCH)], row_v.at[b], sem_i.at[b])
        pltpu.async_copy(col_hbm.at[pl.ds(r0, BCH)], col_v.at[b], sem_i.at[b])
        pltpu.async_copy(tr_hbm.at[pl.ds(r0, BCH)], tr_v.at[b], sem_i.at[b])

    def idx_wait(blk, b):
        r0 = cbase + blk * BCH
        pltpu.make_async_copy(row_hbm.at[pl.ds(r0, BCH)], row_v.at[b], sem_i.at[b]).wait()
        pltpu.make_async_copy(col_hbm.at[pl.ds(r0, BCH)], col_v.at[b], sem_i.at[b]).wait()
        pltpu.make_async_copy(tr_hbm.at[pl.ds(r0, BCH)], tr_v.at[b], sem_i.at[b]).wait()

    def gat_start(ib, jb, g):
        pltpu.async_copy(table.at[row_v.at[ib, jb]], gat_v.at[g], sem_g.at[g])

    def gat_wait(ib, jb, g):
        pltpu.make_async_copy(table.at[row_v.at[ib, jb]], gat_v.at[g], sem_g.at[g]).wait()

    def scat_start(ib, jb, g):
        pltpu.async_copy(gat_v.at[g], acc.at[col_v.at[ib, jb]], sem_s.at[g], add=True)

    def scat_wait(g):
        pltpu.make_async_copy(gat_v.at[g], acc.at[col_v.at[0, 0]], sem_s.at[g]).wait()

    def scale(ib, jb, gb):
        @pl.loop(0, CHUNK // L)
        def _(g):
            t16 = tr_v[ib, jb, pl.ds(g * L, L)]
            for l in range(L):
                e = g * L + l
                t = t16[l]
                for d in range(DH // L):
                    sl = pl.ds(d * L, L)
                    gat_v[gb, e, sl] = gat_v[gb, e, sl] * t

    pltpu.sync_copy(agg_hbm.at[c, pl.ds(s * RPT, RPT)], table.at[pl.ds(s * RPT, RPT)])

    @pl.loop(0, CHUNK)
    def _(r):
        for k in range(DH // L):
            gat_v[0, r, pl.ds(k * L, L)] = jnp.zeros((L,), jnp.float32)

    nz = RPT // CHUNK
    rem = RPT - nz * CHUNK
    for j in range(nz):
        pltpu.sync_copy(gat_v.at[0], acc.at[pl.ds(s * RPT + j * CHUNK, CHUNK)])
    pltpu.sync_copy(gat_v.at[0, pl.ds(0, rem)],
                    acc.at[pl.ds(s * RPT + nz * CHUNK, rem)])
    plsc.subcore_barrier()

    idx_start(0, 0)
    idx_start(1, 1)
    idx_wait(0, 0)
    gat_start(0, 0, 0)

    @pl.loop(0, NCH)
    def _(ch):
        blk = lax.div(ch, BCH)
        jb = lax.rem(ch, BCH)
        ib = lax.rem(blk, 3)
        gb = lax.rem(ch, 2)

        @pl.when(jnp.logical_and(jb == 2, blk + 2 < NBLK))
        def _():
            idx_start(blk + 2, lax.rem(blk + 2, 3))

        @pl.when(jnp.logical_and(ch >= 1, ch + 1 < NCH))
        def _():
            scat_wait(1 - gb)

        @pl.when(ch + 1 < NCH)
        def _():
            blk1 = lax.div(ch + 1, BCH)

            @pl.when(jb == BCH - 1)
            def _():
                idx_wait(blk1, lax.rem(blk1, 3))

            gat_start(lax.rem(blk1, 3), lax.rem(ch + 1, BCH), 1 - gb)

        gat_wait(ib, jb, gb)
        scale(ib, jb, gb)
        scat_start(ib, jb, gb)

    scat_wait(0)
    scat_wait(1)
    plsc.subcore_barrier()
    for j in range(nz):
        rs = s * RPT + j * CHUNK
        pltpu.sync_copy(acc.at[pl.ds(rs, CHUNK)], out_hbm.at[c, pl.ds(rs, CHUNK)])
    rs = s * RPT + nz * CHUNK
    pltpu.sync_copy(acc.at[pl.ds(rs, rem)], out_hbm.at[c, pl.ds(rs, rem)])


_hop = pl.kernel(
    _hop_body,
    out_type=jax.ShapeDtypeStruct((NC, NP, DH), jnp.float32),
    mesh=plsc.VectorSubcoreMesh(core_axis_name="c", subcore_axis_name="s"),
    scratch_types=[
        pltpu.VMEM_SHARED((NP, DH), jnp.float32),
        pltpu.VMEM_SHARED((NP, DH), jnp.float32),
        pltpu.VMEM((3, BCH, CHUNK), jnp.int32),
        pltpu.VMEM((3, BCH, CHUNK), jnp.int32),
        pltpu.VMEM((3, BCH, CHUNK), jnp.float32),
        pltpu.VMEM((2, CHUNK, DH), jnp.float32),
        pltpu.SemaphoreType.DMA((3,)),
        pltpu.SemaphoreType.DMA((2,)),
        pltpu.SemaphoreType.DMA((2,)),
    ],
)


@jax.jit
def kernel(embed, edge_index, trend):
    row = edge_index[0].astype(jnp.int32)
    col = edge_index[1].astype(jnp.int32)
    pad = EP - N_EDGES
    row = jnp.concatenate([row, jnp.zeros((pad,), jnp.int32)]).reshape(-1, CHUNK)
    col = jnp.concatenate([col, jnp.zeros((pad,), jnp.int32)]).reshape(-1, CHUNK)
    tr = jnp.concatenate([trend, jnp.zeros((pad,), jnp.float32)]).reshape(-1, CHUNK)

    npad = NP - N_NODES
    agg2 = jnp.stack([
        jnp.concatenate([embed[:, :DH], jnp.zeros((npad, DH), jnp.float32)]),
        jnp.concatenate([embed[:, DH:], jnp.zeros((npad, DH), jnp.float32)]),
    ])
    embs = [embed]
    for _ in range(N_HOPS):
        agg2 = _hop(agg2, row, col, tr)
        embs.append(jnp.concatenate([agg2[0, :N_NODES], agg2[1, :N_NODES]], axis=1))
    return jnp.stack(embs, axis=1)
